# baseline (speedup 1.0000x reference)
import jax
import jax.numpy as jnp
import numpy as np
from jax import lax
from jax.experimental import pallas as pl
from jax.experimental.pallas import tpu as pltpu
from jax.experimental.pallas import tpu_sc as plsc

N_NODES = 10000
D = 128
DW = D // 2
S = 32
E = 128

NW = 32
NS = 16
PER_W = 320
NPAD = NW * PER_W
NODES_PER_STEP = 4
STEPS = PER_W // NODES_PER_STEP
GROWS = NODES_PER_STEP * S
NBUF = 4
STAGE_ROWS = N_NODES // NS

_HIMASK = np.uint32(0xFFFF0000)
_HALF = np.uint32(0x8000)


def _sc_body(rawp_hbm, nodes_hbm, nidx_hbm, self_hbm, neigh_hbm,
             nidx_v, nodes_v, sbuf_v, grows_v, outbuf_v, tbl_v,
             sem_t, sem_s0, sem_s1, sem_g0, sem_g1, sem_g2, sem_g3):
    cid = lax.axis_index("c")
    sid = lax.axis_index("s")
    wid = sid * 2 + cid
    ssems = (sem_s0, sem_s1)
    gsems = (sem_g0, sem_g1, sem_g2, sem_g3)

    stage = pltpu.async_copy(
        rawp_hbm.at[pl.ds(sid * STAGE_ROWS, STAGE_ROWS)],
        tbl_v.at[pl.ds(sid * STAGE_ROWS, STAGE_ROWS)], sem_t)

    pltpu.sync_copy(nidx_hbm.at[pl.ds(wid * PER_W * S, PER_W * S)], nidx_v)
    pltpu.sync_copy(nodes_hbm.at[pl.ds(wid * PER_W, PER_W)], nodes_v)

    stage.wait()
    plsc.subcore_barrier()

    def g_start(t, b):
        pltpu.async_copy(tbl_v.at[nidx_v.at[pl.ds(t * GROWS, GROWS)]],
                         grows_v.at[b], gsems[b])

    def g_wait(t, b):
        pltpu.make_async_copy(tbl_v.at[nidx_v.at[pl.ds(t * GROWS, GROWS)]],
                              grows_v.at[b], gsems[b]).wait()

    for b in range(NBUF):
        g_start(b, b)

    def s_start(c):
        pltpu.async_copy(tbl_v.at[nodes_v.at[pl.ds(c * 64, 64)]],
                         sbuf_v.at[c % 2], ssems[c % 2])

    s_start(0)
    s_start(1)
    for c in range(5):
        pltpu.make_async_copy(tbl_v.at[nodes_v.at[pl.ds(c * 64, 64)]],
                              sbuf_v.at[c % 2], ssems[c % 2]).wait()
        pltpu.sync_copy(sbuf_v.at[c % 2],
                        self_hbm.at[pl.ds(wid * PER_W + c * 64, 64)])
        if c + 2 < 5:
            s_start(c + 2)

    def loop_body(i, carry):
        for b in range(NBUF):
            s = i * NBUF + b
            g_wait(s, b)
            for n in range(NODES_PER_STEP):
                r0 = n * S

                def load_eo(row, w, b=b):
                    word = grows_v[b, row, pl.ds(16 * w, 16)]
                    e = lax.bitcast_convert_type(word << 16, jnp.float32)
                    o = lax.bitcast_convert_type(word & _HIMASK, jnp.float32)
                    return e, o

                def acc_row(accs, row):
                    a = list(accs)
                    for w in range(4):
                        e, o = load_eo(row, w)
                        a[2 * w] = a[2 * w] + e
                        a[2 * w + 1] = a[2 * w + 1] + o
                    return tuple(a)

                def jbody(jj, accs, r0=r0):
                    accs = acc_row(accs, r0 + jj * 2)
                    return acc_row(accs, r0 + jj * 2 + 1)

                accs = []
                for w in range(4):
                    e, o = load_eo(r0, w)
                    accs.extend((e, o))
                accs = acc_row(tuple(accs), r0 + 1)
                accs = lax.fori_loop(1, S // 2, jbody, accs)
                row = s * NODES_PER_STEP + n
                for w in range(4):
                    e_bits = lax.bitcast_convert_type(
                        accs[2 * w] * (1.0 / S), jnp.uint32)
                    o_bits = lax.bitcast_convert_type(
                        accs[2 * w + 1] * (1.0 / S), jnp.uint32)
                    outbuf_v[row, pl.ds(16 * w, 16)] = (
                        ((e_bits + _HALF) >> 16)
                        | ((o_bits + _HALF) & _HIMASK))
            nxt = s + NBUF
            pl.when(nxt < STEPS)(lambda t=nxt, bb=b: g_start(t, bb))
        return carry

    lax.fori_loop(0, STEPS // NBUF, loop_body, 0)

    pltpu.sync_copy(outbuf_v, neigh_hbm.at[pl.ds(wid * PER_W, PER_W)])


def _pack_body(x_ref, o_ref):
    bits = lax.bitcast_convert_type(x_ref[...], jnp.uint32)
    o_ref[...] = (((bits[:, :DW] + _HALF) >> 16)
                  | ((bits[:, DW:] + _HALF) & _HIMASK))


def _mm_body(wse_ref, wso_ref, wne_ref, wno_ref, s_ref, n_ref, o_ref):
    def half(words, we_ref, wo_ref):
        e = lax.bitcast_convert_type(words << 16, jnp.float32)
        o = lax.bitcast_convert_type(words & _HIMASK, jnp.float32)
        ct = (((1,), (1,)), ((), ()))
        return (lax.dot_general(we_ref[...], e.astype(jnp.bfloat16), ct,
                                preferred_element_type=jnp.float32)
                + lax.dot_general(wo_ref[...], o.astype(jnp.bfloat16), ct,
                                  preferred_element_type=jnp.float32))

    pre = (half(s_ref[...], wse_ref, wso_ref)
           + half(n_ref[...], wne_ref, wno_ref))
    o_ref[...] = jnp.where(pre >= 0, pre, 0.01 * pre)


def kernel(raw_features, nodes, neigh_index, weight):
    pad = NPAD - N_NODES
    nodes_p = jnp.concatenate([nodes, jnp.zeros((pad,), jnp.int32)])
    nidx_p = jnp.concatenate(
        [neigh_index.reshape(-1), jnp.zeros((pad * S,), jnp.int32)])
    raw_packed = pl.pallas_call(
        _pack_body,
        grid=(10,),
        in_specs=[pl.BlockSpec((N_NODES // 10, D), lambda i: (i, 0))],
        out_specs=pl.BlockSpec((N_NODES // 10, DW), lambda i: (i, 0)),
        out_shape=jax.ShapeDtypeStruct((N_NODES, DW), jnp.uint32),
    )(raw_features)

    mesh = plsc.VectorSubcoreMesh(core_axis_name="c", subcore_axis_name="s")
    sc_gather = pl.kernel(
        _sc_body,
        out_type=(jax.ShapeDtypeStruct((NPAD, DW), jnp.uint32),
                  jax.ShapeDtypeStruct((NPAD, DW), jnp.uint32)),
        mesh=mesh,
        compiler_params=pltpu.CompilerParams(use_tc_tiling_on_sc=False),
        scratch_types=[
            pltpu.VMEM((PER_W * S,), jnp.int32),
            pltpu.VMEM((PER_W,), jnp.int32),
            pltpu.VMEM((2, 64, DW), jnp.uint32),
            pltpu.VMEM((NBUF, GROWS, DW), jnp.uint32),
            pltpu.VMEM((PER_W, DW), jnp.uint32),
            pltpu.VMEM_SHARED((N_NODES, DW), jnp.uint32),
            pltpu.SemaphoreType.DMA,
            pltpu.SemaphoreType.DMA,
            pltpu.SemaphoreType.DMA,
            pltpu.SemaphoreType.DMA,
            pltpu.SemaphoreType.DMA,
            pltpu.SemaphoreType.DMA,
            pltpu.SemaphoreType.DMA,
        ],
    )
    self_packed, neigh_packed = sc_gather(raw_packed, nodes_p, nidx_p)

    wse = weight[:, 0:DW].astype(jnp.bfloat16)
    wso = weight[:, DW:D].astype(jnp.bfloat16)
    wne = weight[:, D:D + DW].astype(jnp.bfloat16)
    wno = weight[:, D + DW:].astype(jnp.bfloat16)
    nb = 1024
    grid = NPAD // nb
    out = pl.pallas_call(
        _mm_body,
        grid=(grid,),
        in_specs=[
            pl.BlockSpec((E, DW), lambda i: (0, 0)),
            pl.BlockSpec((E, DW), lambda i: (0, 0)),
            pl.BlockSpec((E, DW), lambda i: (0, 0)),
            pl.BlockSpec((E, DW), lambda i: (0, 0)),
            pl.BlockSpec((nb, DW), lambda i: (i, 0)),
            pl.BlockSpec((nb, DW), lambda i: (i, 0)),
        ],
        out_specs=pl.BlockSpec((E, nb), lambda i: (0, i)),
        out_shape=jax.ShapeDtypeStruct((E, N_NODES), jnp.float32),
    )(wse, wso, wne, wno, self_packed, neigh_packed)
    return out


# speedup vs baseline: 1.0755x; 1.0755x over previous
import jax
import jax.numpy as jnp
import numpy as np
from jax import lax
from jax.experimental import pallas as pl
from jax.experimental.pallas import tpu as pltpu
from jax.experimental.pallas import tpu_sc as plsc

N_NODES = 10000
D = 128
DW = D // 2
S = 32
E = 128

NW = 32
NS = 16
PER_W = 320
NPAD = NW * PER_W
NODES_PER_STEP = 4
STEPS = PER_W // NODES_PER_STEP
GROWS = NODES_PER_STEP * S
NBUF = 4
STAGE_ROWS = N_NODES // NS

_HIMASK = np.uint32(0xFFFF0000)
_HALF = np.uint32(0x8000)


def _sc_body(rawp_hbm, nodes_hbm, nidx_hbm, self_hbm, neigh_hbm,
             nidx_v, nodes_v, sbuf_v, grows_v, outbuf_v, tbl_v,
             sem_t, sem_s0, sem_s1, sem_g0, sem_g1, sem_g2, sem_g3):
    cid = lax.axis_index("c")
    sid = lax.axis_index("s")
    wid = sid * 2 + cid
    ssems = (sem_s0, sem_s1)
    gsems = (sem_g0, sem_g1, sem_g2, sem_g3)

    stage = pltpu.async_copy(
        rawp_hbm.at[pl.ds(sid * STAGE_ROWS, STAGE_ROWS)],
        tbl_v.at[pl.ds(sid * STAGE_ROWS, STAGE_ROWS)], sem_t)

    pltpu.sync_copy(nidx_hbm.at[pl.ds(wid * PER_W * S, PER_W * S)], nidx_v)
    pltpu.sync_copy(nodes_hbm.at[pl.ds(wid * PER_W, PER_W)], nodes_v)

    stage.wait()
    plsc.subcore_barrier()

    def g_start(t, b):
        pltpu.async_copy(tbl_v.at[nidx_v.at[pl.ds(t * GROWS, GROWS)]],
                         grows_v.at[b], gsems[b])

    def g_wait(t, b):
        pltpu.make_async_copy(tbl_v.at[nidx_v.at[pl.ds(t * GROWS, GROWS)]],
                              grows_v.at[b], gsems[b]).wait()

    for b in range(NBUF):
        g_start(b, b)

    def s_start(c):
        pltpu.async_copy(tbl_v.at[nodes_v.at[pl.ds(c * 64, 64)]],
                         sbuf_v.at[c % 2], ssems[c % 2])

    s_start(0)
    s_start(1)
    for c in range(5):
        pltpu.make_async_copy(tbl_v.at[nodes_v.at[pl.ds(c * 64, 64)]],
                              sbuf_v.at[c % 2], ssems[c % 2]).wait()
        pltpu.sync_copy(sbuf_v.at[c % 2],
                        self_hbm.at[pl.ds(wid * PER_W + c * 64, 64)])
        if c + 2 < 5:
            s_start(c + 2)

    def loop_body(i, carry):
        for b in range(NBUF):
            s = i * NBUF + b
            g_wait(s, b)
            for n in range(NODES_PER_STEP):
                r0 = n * S

                def load_eo(row, w, b=b):
                    word = grows_v[b, row, pl.ds(16 * w, 16)]
                    e = lax.bitcast_convert_type(word << 16, jnp.float32)
                    o = lax.bitcast_convert_type(word & _HIMASK, jnp.float32)
                    return e, o

                def acc_row(accs, row):
                    a = list(accs)
                    for w in range(4):
                        e, o = load_eo(row, w)
                        a[2 * w] = a[2 * w] + e
                        a[2 * w + 1] = a[2 * w + 1] + o
                    return tuple(a)

                def jbody(jj, accs, r0=r0):
                    accs = acc_row(accs, r0 + jj * 2)
                    return acc_row(accs, r0 + jj * 2 + 1)

                accs = []
                for w in range(4):
                    e, o = load_eo(r0, w)
                    accs.extend((e, o))
                accs = acc_row(tuple(accs), r0 + 1)
                accs = lax.fori_loop(1, S // 2, jbody, accs)
                row = s * NODES_PER_STEP + n
                for w in range(4):
                    e_bits = lax.bitcast_convert_type(
                        accs[2 * w] * (1.0 / S), jnp.uint32)
                    o_bits = lax.bitcast_convert_type(
                        accs[2 * w + 1] * (1.0 / S), jnp.uint32)
                    outbuf_v[row, pl.ds(16 * w, 16)] = (
                        ((e_bits + _HALF) >> 16)
                        | ((o_bits + _HALF) & _HIMASK))
            nxt = s + NBUF
            pl.when(nxt < STEPS)(lambda t=nxt, bb=b: g_start(t, bb))
        return carry

    lax.fori_loop(0, STEPS // NBUF, loop_body, 0)

    pltpu.sync_copy(outbuf_v, neigh_hbm.at[pl.ds(wid * PER_W, PER_W)])


def _mm_body(wse_ref, wso_ref, wne_ref, wno_ref, s_ref, n_ref, o_ref):
    def half(words, we_ref, wo_ref):
        e = lax.bitcast_convert_type(words << 16, jnp.float32)
        o = lax.bitcast_convert_type(words & _HIMASK, jnp.float32)
        ct = (((1,), (1,)), ((), ()))
        return (lax.dot_general(we_ref[...], e.astype(jnp.bfloat16), ct,
                                preferred_element_type=jnp.float32)
                + lax.dot_general(wo_ref[...], o.astype(jnp.bfloat16), ct,
                                  preferred_element_type=jnp.float32))

    pre = (half(s_ref[...], wse_ref, wso_ref)
           + half(n_ref[...], wne_ref, wno_ref))
    o_ref[...] = jnp.where(pre >= 0, pre, 0.01 * pre)


def kernel(raw_features, nodes, neigh_index, weight):
    pad = NPAD - N_NODES
    nodes_p = jnp.concatenate([nodes, jnp.zeros((pad,), jnp.int32)])
    nidx_p = jnp.concatenate(
        [neigh_index.reshape(-1), jnp.zeros((pad * S,), jnp.int32)])
    bits = lax.bitcast_convert_type(raw_features, jnp.uint32)
    raw_packed = (((bits[:, :DW] + _HALF) >> 16)
                  | ((bits[:, DW:] + _HALF) & _HIMASK))

    mesh = plsc.VectorSubcoreMesh(core_axis_name="c", subcore_axis_name="s")
    sc_gather = pl.kernel(
        _sc_body,
        out_type=(jax.ShapeDtypeStruct((NPAD, DW), jnp.uint32),
                  jax.ShapeDtypeStruct((NPAD, DW), jnp.uint32)),
        mesh=mesh,
        compiler_params=pltpu.CompilerParams(use_tc_tiling_on_sc=False),
        scratch_types=[
            pltpu.VMEM((PER_W * S,), jnp.int32),
            pltpu.VMEM((PER_W,), jnp.int32),
            pltpu.VMEM((2, 64, DW), jnp.uint32),
            pltpu.VMEM((NBUF, GROWS, DW), jnp.uint32),
            pltpu.VMEM((PER_W, DW), jnp.uint32),
            pltpu.VMEM_SHARED((N_NODES, DW), jnp.uint32),
            pltpu.SemaphoreType.DMA,
            pltpu.SemaphoreType.DMA,
            pltpu.SemaphoreType.DMA,
            pltpu.SemaphoreType.DMA,
            pltpu.SemaphoreType.DMA,
            pltpu.SemaphoreType.DMA,
            pltpu.SemaphoreType.DMA,
        ],
    )
    self_packed, neigh_packed = sc_gather(raw_packed, nodes_p, nidx_p)

    wse = weight[:, 0:DW].astype(jnp.bfloat16)
    wso = weight[:, DW:D].astype(jnp.bfloat16)
    wne = weight[:, D:D + DW].astype(jnp.bfloat16)
    wno = weight[:, D + DW:].astype(jnp.bfloat16)
    nb = 1024
    grid = NPAD // nb
    out = pl.pallas_call(
        _mm_body,
        grid=(grid,),
        in_specs=[
            pl.BlockSpec((E, DW), lambda i: (0, 0)),
            pl.BlockSpec((E, DW), lambda i: (0, 0)),
            pl.BlockSpec((E, DW), lambda i: (0, 0)),
            pl.BlockSpec((E, DW), lambda i: (0, 0)),
            pl.BlockSpec((nb, DW), lambda i: (i, 0)),
            pl.BlockSpec((nb, DW), lambda i: (i, 0)),
        ],
        out_specs=pl.BlockSpec((E, nb), lambda i: (0, i)),
        out_shape=jax.ShapeDtypeStruct((E, N_NODES), jnp.float32),
    )(wse, wso, wne, wno, self_packed, neigh_packed)
    return out


# speedup vs baseline: 1.0879x; 1.0116x over previous
import jax
import jax.numpy as jnp
import numpy as np
from jax import lax
from jax.experimental import pallas as pl
from jax.experimental.pallas import tpu as pltpu
from jax.experimental.pallas import tpu_sc as plsc

N_NODES = 10000
D = 128
DW = D // 2
S = 32
E = 128

NW = 32
NS = 16
PER_W = 320
NPAD = NW * PER_W
NODES_PER_STEP = 4
STEPS = PER_W // NODES_PER_STEP
GROWS = NODES_PER_STEP * S
NBUF = 4
STAGE_ROWS = N_NODES // NS

_HIMASK = np.uint32(0xFFFF0000)
_HALF = np.uint32(0x8000)


def _sc_body(rawp_hbm, nodes_hbm, nidx_hbm, self_hbm, neigh_hbm,
             nidx_v, nodes_v, sbuf_v, grows_v, outbuf_v, tbl_v,
             sem_t, sem_s0, sem_g0, sem_g1, sem_g2, sem_g3):
    cid = lax.axis_index("c")
    sid = lax.axis_index("s")
    wid = sid * 2 + cid
    gsems = (sem_g0, sem_g1, sem_g2, sem_g3)

    stage = pltpu.async_copy(
        rawp_hbm.at[pl.ds(sid * STAGE_ROWS, STAGE_ROWS)],
        tbl_v.at[pl.ds(sid * STAGE_ROWS, STAGE_ROWS)], sem_t)

    pltpu.sync_copy(nidx_hbm.at[pl.ds(wid * PER_W * S, PER_W * S)], nidx_v)
    pltpu.sync_copy(nodes_hbm.at[pl.ds(wid * PER_W, PER_W)], nodes_v)

    stage.wait()
    plsc.subcore_barrier()

    def g_start(t, b):
        pltpu.async_copy(tbl_v.at[nidx_v.at[pl.ds(t * GROWS, GROWS)]],
                         grows_v.at[b], gsems[b])

    def g_wait(t, b):
        pltpu.make_async_copy(tbl_v.at[nidx_v.at[pl.ds(t * GROWS, GROWS)]],
                              grows_v.at[b], gsems[b]).wait()

    for b in range(NBUF):
        g_start(b, b)

    for c in range(5):
        pltpu.async_copy(tbl_v.at[nodes_v.at[pl.ds(c * 64, 64)]],
                         sbuf_v.at[pl.ds(c * 64, 64)], sem_s0)

    def loop_body(i, carry):
        for b in range(NBUF):
            s = i * NBUF + b
            g_wait(s, b)
            for n in range(NODES_PER_STEP):
                r0 = n * S

                def load_eo(row, w, b=b):
                    word = grows_v[b, row, pl.ds(16 * w, 16)]
                    e = lax.bitcast_convert_type(word << 16, jnp.float32)
                    o = lax.bitcast_convert_type(word & _HIMASK, jnp.float32)
                    return e, o

                def acc_row(accs, row):
                    a = list(accs)
                    for w in range(4):
                        e, o = load_eo(row, w)
                        a[2 * w] = a[2 * w] + e
                        a[2 * w + 1] = a[2 * w + 1] + o
                    return tuple(a)

                def jbody(jj, accs, r0=r0):
                    accs = acc_row(accs, r0 + jj * 2)
                    return acc_row(accs, r0 + jj * 2 + 1)

                accs = []
                for w in range(4):
                    e, o = load_eo(r0, w)
                    accs.extend((e, o))
                accs = acc_row(tuple(accs), r0 + 1)
                accs = lax.fori_loop(1, S // 2, jbody, accs)
                row = s * NODES_PER_STEP + n
                for w in range(4):
                    e_bits = lax.bitcast_convert_type(
                        accs[2 * w] * (1.0 / S), jnp.uint32)
                    o_bits = lax.bitcast_convert_type(
                        accs[2 * w + 1] * (1.0 / S), jnp.uint32)
                    outbuf_v[row, pl.ds(16 * w, 16)] = (
                        ((e_bits + _HALF) >> 16)
                        | ((o_bits + _HALF) & _HIMASK))
            nxt = s + NBUF
            pl.when(nxt < STEPS)(lambda t=nxt, bb=b: g_start(t, bb))
        return carry

    lax.fori_loop(0, STEPS // NBUF, loop_body, 0)

    pltpu.sync_copy(outbuf_v, neigh_hbm.at[pl.ds(wid * PER_W, PER_W)])
    for c in range(5):
        pltpu.make_async_copy(tbl_v.at[nodes_v.at[pl.ds(c * 64, 64)]],
                              sbuf_v.at[pl.ds(c * 64, 64)], sem_s0).wait()
    pltpu.sync_copy(sbuf_v, self_hbm.at[pl.ds(wid * PER_W, PER_W)])


def _mm_body(wse_ref, wso_ref, wne_ref, wno_ref, s_ref, n_ref, o_ref):
    def half(words, we_ref, wo_ref):
        e = lax.bitcast_convert_type(words << 16, jnp.float32)
        o = lax.bitcast_convert_type(words & _HIMASK, jnp.float32)
        ct = (((1,), (1,)), ((), ()))
        return (lax.dot_general(we_ref[...], e.astype(jnp.bfloat16), ct,
                                preferred_element_type=jnp.float32)
                + lax.dot_general(wo_ref[...], o.astype(jnp.bfloat16), ct,
                                  preferred_element_type=jnp.float32))

    pre = (half(s_ref[...], wse_ref, wso_ref)
           + half(n_ref[...], wne_ref, wno_ref))
    o_ref[...] = jnp.where(pre >= 0, pre, 0.01 * pre)


def kernel(raw_features, nodes, neigh_index, weight):
    pad = NPAD - N_NODES
    nodes_p = jnp.concatenate([nodes, jnp.zeros((pad,), jnp.int32)])
    nidx_p = jnp.concatenate(
        [neigh_index.reshape(-1), jnp.zeros((pad * S,), jnp.int32)])
    bits = lax.bitcast_convert_type(raw_features, jnp.uint32)
    raw_packed = (((bits[:, :DW] + _HALF) >> 16)
                  | ((bits[:, DW:] + _HALF) & _HIMASK))

    mesh = plsc.VectorSubcoreMesh(core_axis_name="c", subcore_axis_name="s")
    sc_gather = pl.kernel(
        _sc_body,
        out_type=(jax.ShapeDtypeStruct((NPAD, DW), jnp.uint32),
                  jax.ShapeDtypeStruct((NPAD, DW), jnp.uint32)),
        mesh=mesh,
        compiler_params=pltpu.CompilerParams(use_tc_tiling_on_sc=False),
        scratch_types=[
            pltpu.VMEM((PER_W * S,), jnp.int32),
            pltpu.VMEM((PER_W,), jnp.int32),
            pltpu.VMEM((PER_W, DW), jnp.uint32),
            pltpu.VMEM((NBUF, GROWS, DW), jnp.uint32),
            pltpu.VMEM((PER_W, DW), jnp.uint32),
            pltpu.VMEM_SHARED((N_NODES, DW), jnp.uint32),
            pltpu.SemaphoreType.DMA,
            pltpu.SemaphoreType.DMA,
            pltpu.SemaphoreType.DMA,
            pltpu.SemaphoreType.DMA,
            pltpu.SemaphoreType.DMA,
            pltpu.SemaphoreType.DMA,
        ],
    )
    self_packed, neigh_packed = sc_gather(raw_packed, nodes_p, nidx_p)

    wse = weight[:, 0:DW].astype(jnp.bfloat16)
    wso = weight[:, DW:D].astype(jnp.bfloat16)
    wne = weight[:, D:D + DW].astype(jnp.bfloat16)
    wno = weight[:, D + DW:].astype(jnp.bfloat16)
    nb = 1024
    grid = NPAD // nb
    out = pl.pallas_call(
        _mm_body,
        grid=(grid,),
        in_specs=[
            pl.BlockSpec((E, DW), lambda i: (0, 0)),
            pl.BlockSpec((E, DW), lambda i: (0, 0)),
            pl.BlockSpec((E, DW), lambda i: (0, 0)),
            pl.BlockSpec((E, DW), lambda i: (0, 0)),
            pl.BlockSpec((nb, DW), lambda i: (i, 0)),
            pl.BlockSpec((nb, DW), lambda i: (i, 0)),
        ],
        out_specs=pl.BlockSpec((E, nb), lambda i: (0, i)),
        out_shape=jax.ShapeDtypeStruct((E, N_NODES), jnp.float32),
    )(wse, wso, wne, wno, self_packed, neigh_packed)
    return out
